# baseline (speedup 1.0000x reference)
import functools

import jax
import jax.numpy as jnp
from jax import lax
from jax.experimental import pallas as pl
from jax.experimental.pallas import tpu as pltpu
from jax.experimental.pallas import tpu_sc as plsc

N = 10000
D = 128
E = 320000
EPS = 1e-5
NC, NS = 2, 16
NTILES = NC * NS
CHUNK = 128
CPT0 = 64
CPT1 = 96
EPT0 = CPT0 * CHUNK
EPT1 = CPT1 * CHUNK
E_PAD = NS * (EPT0 + EPT1)
NPAD = 10112
ROWS_PT = NPAD // NS

_MESH = plsc.VectorSubcoreMesh(core_axis_name="c", subcore_axis_name="s")


def _run_edge_loop(c, s, body_for_chunk):
    @pl.when(c == 0)
    def _():
        base = s * EPT0

        def step(j, carry):
            body_for_chunk(base + j * CHUNK)
            return carry

        lax.fori_loop(0, CPT0, step, 0)

    @pl.when(c == 1)
    def _():
        base = NS * EPT0 + s * EPT1

        def step(j, carry):
            body_for_chunk(base + j * CHUNK)
            return carry

        lax.fori_loop(0, CPT1, step, 0)


def _deg_body(dst_hbm, ones_hbm, out_hbm, dst_v, ones_v, zbuf_v, deg_sh):
    c = lax.axis_index("c")
    s = lax.axis_index("s")

    pltpu.sync_copy(ones_hbm.at[pl.ds(0, CHUNK)], ones_v)

    def zloop(k, carry):
        off = s * ROWS_PT + k * CHUNK
        pltpu.sync_copy(zbuf_v, deg_sh.at[pl.ds(off, CHUNK)])
        return carry

    pltpu.sync_copy(ones_hbm.at[pl.ds(CHUNK, CHUNK)], zbuf_v)
    lax.fori_loop(0, ROWS_PT // CHUNK, zloop, 0)
    tail = ROWS_PT % CHUNK
    toff = s * ROWS_PT + (ROWS_PT // CHUNK) * CHUNK
    pltpu.sync_copy(zbuf_v.at[pl.ds(0, tail)], deg_sh.at[pl.ds(toff, tail)])
    plsc.subcore_barrier()

    def chunk_body(eoff):
        pltpu.sync_copy(dst_hbm.at[pl.ds(eoff, CHUNK)], dst_v)
        pltpu.sync_copy(ones_v, deg_sh.at[dst_v], add=True)

    _run_edge_loop(c, s, chunk_body)
    plsc.subcore_barrier()

    def writeback(k, carry):
        off = s * ROWS_PT + k * CHUNK
        pltpu.sync_copy(deg_sh.at[pl.ds(off, CHUNK)], zbuf_v)
        pltpu.sync_copy(zbuf_v, out_hbm.at[pl.ds(c * NPAD + off, CHUNK)])
        return carry

    lax.fori_loop(0, ROWS_PT // CHUNK, writeback, 0)
    pltpu.sync_copy(deg_sh.at[pl.ds(toff, tail)], zbuf_v.at[pl.ds(0, tail)])
    pltpu.sync_copy(zbuf_v.at[pl.ds(0, tail)], out_hbm.at[pl.ds(c * NPAD + toff, tail)])


_deg_call = pl.kernel(
    _deg_body,
    out_type=jax.ShapeDtypeStruct((NC * NPAD, D), jnp.float32),
    mesh=_MESH,
    scratch_types=[
        pltpu.VMEM((CHUNK,), jnp.int32),
        pltpu.VMEM((CHUNK, D), jnp.float32),
        pltpu.VMEM((CHUNK, D), jnp.float32),
        pltpu.VMEM_SHARED((NPAD, D), jnp.float32),
    ],
)


def _agg_body(hn_hbm, src_hbm, dst_hbm, out_hbm, src_v, dst_v, rows_v,
              agg_sh, sem):
    c = lax.axis_index("c")
    s = lax.axis_index("s")

    def stage(k, carry):
        off = s * ROWS_PT + k * CHUNK
        pltpu.sync_copy(hn_hbm.at[pl.ds(off, CHUNK)], rows_v)
        pltpu.sync_copy(rows_v, agg_sh.at[pl.ds(off, CHUNK)])
        return carry

    lax.fori_loop(0, ROWS_PT // CHUNK, stage, 0)
    tail = ROWS_PT % CHUNK
    toff = s * ROWS_PT + (ROWS_PT // CHUNK) * CHUNK
    pltpu.sync_copy(hn_hbm.at[pl.ds(toff, tail)], rows_v.at[pl.ds(0, tail)])
    pltpu.sync_copy(rows_v.at[pl.ds(0, tail)], agg_sh.at[pl.ds(toff, tail)])
    plsc.subcore_barrier()

    def chunk_body(eoff):
        pltpu.sync_copy(src_hbm.at[pl.ds(eoff, CHUNK)], src_v)
        pltpu.sync_copy(dst_hbm.at[pl.ds(eoff, CHUNK)], dst_v)
        pltpu.async_copy(hn_hbm.at[src_v], rows_v, sem).wait()
        pltpu.sync_copy(rows_v, agg_sh.at[dst_v], add=True)

    _run_edge_loop(c, s, chunk_body)
    plsc.subcore_barrier()

    def writeback(k, carry):
        off = s * ROWS_PT + k * CHUNK
        pltpu.sync_copy(agg_sh.at[pl.ds(off, CHUNK)], rows_v)
        pltpu.sync_copy(rows_v, out_hbm.at[pl.ds(c * NPAD + off, CHUNK)])
        return carry

    lax.fori_loop(0, ROWS_PT // CHUNK, writeback, 0)
    pltpu.sync_copy(agg_sh.at[pl.ds(toff, tail)], rows_v.at[pl.ds(0, tail)])
    pltpu.sync_copy(rows_v.at[pl.ds(0, tail)], out_hbm.at[pl.ds(c * NPAD + toff, tail)])


_agg_call = pl.kernel(
    _agg_body,
    out_type=jax.ShapeDtypeStruct((NC * NPAD, D), jnp.float32),
    mesh=_MESH,
    scratch_types=[
        pltpu.VMEM((CHUNK,), jnp.int32),
        pltpu.VMEM((CHUNK,), jnp.int32),
        pltpu.VMEM((CHUNK, D), jnp.float32),
        pltpu.VMEM_SHARED((NPAD, D), jnp.float32),
        pltpu.SemaphoreType.DMA,
    ],
)


def _lin_body(x_ref, w_ref, d0_ref, d1_ref, hn_ref):
    deg = d0_ref[...] + d1_ref[...] + 1.0
    norm = lax.rsqrt(deg)
    h = lax.dot_general(
        x_ref[...], w_ref[...], (((1,), (1,)), ((), ())),
        preferred_element_type=jnp.float32,
    )
    hn_ref[...] = h * norm


ROWS_B = 1000

_lin_call = pl.pallas_call(
    _lin_body,
    grid=(N // ROWS_B,),
    in_specs=[
        pl.BlockSpec((ROWS_B, D), lambda i: (i, 0)),
        pl.BlockSpec((D, D), lambda i: (0, 0)),
        pl.BlockSpec((ROWS_B, 1), lambda i: (i, 0)),
        pl.BlockSpec((ROWS_B, 1), lambda i: (i, 0)),
    ],
    out_specs=pl.BlockSpec((ROWS_B, D), lambda i: (i, 0)),
    out_shape=jax.ShapeDtypeStruct((N, D), jnp.float32),
)


def _ln_body(p0_ref, p1_ref, hn_ref, x_ref, d0_ref, d1_ref, g_ref, b_ref, o_ref):
    deg = d0_ref[...] + d1_ref[...] + 1.0
    norm = lax.rsqrt(deg)
    agg = (p0_ref[...] + p1_ref[...] - hn_ref[...]) * norm
    h = agg + x_ref[...]
    mean = jnp.mean(h, axis=-1, keepdims=True)
    cent = h - mean
    var = jnp.mean(cent * cent, axis=-1, keepdims=True)
    hln = cent * lax.rsqrt(var + EPS) * g_ref[0:1, :] + b_ref[0:1, :]
    o_ref[...] = jnp.maximum(hln, 0.0)


_ln_call = pl.pallas_call(
    _ln_body,
    grid=(N // ROWS_B,),
    in_specs=[
        pl.BlockSpec((ROWS_B, D), lambda i: (i, 0)),
        pl.BlockSpec((ROWS_B, D), lambda i: (i, 0)),
        pl.BlockSpec((ROWS_B, D), lambda i: (i, 0)),
        pl.BlockSpec((ROWS_B, D), lambda i: (i, 0)),
        pl.BlockSpec((ROWS_B, 1), lambda i: (i, 0)),
        pl.BlockSpec((ROWS_B, 1), lambda i: (i, 0)),
        pl.BlockSpec((8, D), lambda i: (0, 0)),
        pl.BlockSpec((8, D), lambda i: (0, 0)),
    ],
    out_specs=pl.BlockSpec((ROWS_B, D), lambda i: (i, 0)),
    out_shape=jax.ShapeDtypeStruct((N, D), jnp.float32),
)


@jax.jit
def kernel(x, edge_index, W, ln_gamma, ln_beta):
    ei = edge_index.astype(jnp.int32)
    pad = jnp.full((E_PAD - E,), N, jnp.int32)
    src_p = jnp.concatenate([ei[0], pad])
    dst_p = jnp.concatenate([ei[1], pad])

    ones_c = jnp.concatenate([
        jnp.ones((CHUNK, D), jnp.float32),
        jnp.zeros((CHUNK, D), jnp.float32),
    ])
    deg_parts = _deg_call(dst_p, ones_c)
    d0 = deg_parts[:N, 0:1]
    d1 = deg_parts[NPAD:NPAD + N, 0:1]

    hn = _lin_call(x, W, d0, d1)
    hn_pad = jnp.concatenate([hn, jnp.zeros((NPAD - N, D), jnp.float32)])

    parts = _agg_call(hn_pad, src_p, dst_p)

    g8 = jnp.broadcast_to(ln_gamma.reshape(1, D), (8, D))
    b8 = jnp.broadcast_to(ln_beta.reshape(1, D), (8, D))
    return _ln_call(parts[:N], parts[NPAD:NPAD + N], hn, x, d0, d1, g8, b8)


# speedup vs baseline: 1.1224x; 1.1224x over previous
import functools

import jax
import jax.numpy as jnp
from jax import lax
from jax.experimental import pallas as pl
from jax.experimental.pallas import tpu as pltpu
from jax.experimental.pallas import tpu_sc as plsc

N = 10000
D = 128
E = 320000
EPS = 1e-5
NC, NS = 2, 16
NTILES = NC * NS
CHUNK = 128
CPT0 = 92
CPT1 = 68
CPT_D = 80
EPT0 = CPT0 * CHUNK
EPT1 = CPT1 * CHUNK
E_PAD = NS * (EPT0 + EPT1)
NPAD = 10112
ROWS_PT = NPAD // NS

_MESH = plsc.VectorSubcoreMesh(core_axis_name="c", subcore_axis_name="s")


def _run_edge_loop(c, s, body_for_chunk):
    @pl.when(c == 0)
    def _():
        base = s * EPT0

        def step(j, carry):
            body_for_chunk(base + j * CHUNK)
            return carry

        lax.fori_loop(0, CPT0, step, 0)

    @pl.when(c == 1)
    def _():
        base = NS * EPT0 + s * EPT1

        def step(j, carry):
            body_for_chunk(base + j * CHUNK)
            return carry

        lax.fori_loop(0, CPT1, step, 0)


def _deg_body(dst_hbm, ones_hbm, out_hbm, dst_v, ones_v, zbuf_v, deg_sh):
    c = lax.axis_index("c")
    s = lax.axis_index("s")

    pltpu.sync_copy(ones_hbm.at[pl.ds(0, CHUNK)], ones_v)

    def zloop(k, carry):
        off = s * ROWS_PT + k * CHUNK
        pltpu.sync_copy(zbuf_v, deg_sh.at[pl.ds(off, CHUNK)])
        return carry

    pltpu.sync_copy(ones_hbm.at[pl.ds(CHUNK, CHUNK)], zbuf_v)
    lax.fori_loop(0, ROWS_PT // CHUNK, zloop, 0)
    tail = ROWS_PT % CHUNK
    toff = s * ROWS_PT + (ROWS_PT // CHUNK) * CHUNK
    pltpu.sync_copy(zbuf_v.at[pl.ds(0, tail)], deg_sh.at[pl.ds(toff, tail)])
    plsc.subcore_barrier()

    base = (c * NS + s) * CPT_D * CHUNK

    def step(j, carry):
        pltpu.sync_copy(dst_hbm.at[pl.ds(base + j * CHUNK, CHUNK)], dst_v)
        pltpu.sync_copy(ones_v, deg_sh.at[dst_v], add=True)
        return carry

    lax.fori_loop(0, CPT_D, step, 0)
    plsc.subcore_barrier()

    def writeback(k, carry):
        off = s * ROWS_PT + k * CHUNK
        pltpu.sync_copy(deg_sh.at[pl.ds(off, CHUNK)], zbuf_v)
        pltpu.sync_copy(zbuf_v, out_hbm.at[pl.ds(c * NPAD + off, CHUNK)])
        return carry

    lax.fori_loop(0, ROWS_PT // CHUNK, writeback, 0)
    pltpu.sync_copy(deg_sh.at[pl.ds(toff, tail)], zbuf_v.at[pl.ds(0, tail)])
    pltpu.sync_copy(zbuf_v.at[pl.ds(0, tail)], out_hbm.at[pl.ds(c * NPAD + toff, tail)])


_deg_call = pl.kernel(
    _deg_body,
    out_type=jax.ShapeDtypeStruct((NC * NPAD, D), jnp.float32),
    mesh=_MESH,
    scratch_types=[
        pltpu.VMEM((CHUNK,), jnp.int32),
        pltpu.VMEM((CHUNK, D), jnp.float32),
        pltpu.VMEM((CHUNK, D), jnp.float32),
        pltpu.VMEM_SHARED((NPAD, D), jnp.float32),
    ],
)


def _agg_body(hn_hbm, src_hbm, dst_hbm, out_hbm, src_v, dst_v, rows_v,
              agg_sh, sem):
    c = lax.axis_index("c")
    s = lax.axis_index("s")

    def stage(k, carry):
        off = s * ROWS_PT + k * CHUNK
        pltpu.sync_copy(hn_hbm.at[pl.ds(off, CHUNK)], rows_v)
        pltpu.sync_copy(rows_v, agg_sh.at[pl.ds(off, CHUNK)])
        return carry

    lax.fori_loop(0, ROWS_PT // CHUNK, stage, 0)
    tail = ROWS_PT % CHUNK
    toff = s * ROWS_PT + (ROWS_PT // CHUNK) * CHUNK
    pltpu.sync_copy(hn_hbm.at[pl.ds(toff, tail)], rows_v.at[pl.ds(0, tail)])
    pltpu.sync_copy(rows_v.at[pl.ds(0, tail)], agg_sh.at[pl.ds(toff, tail)])
    plsc.subcore_barrier()

    def chunk_body(eoff):
        pltpu.sync_copy(src_hbm.at[pl.ds(eoff, CHUNK)], src_v)
        pltpu.sync_copy(dst_hbm.at[pl.ds(eoff, CHUNK)], dst_v)
        pltpu.async_copy(hn_hbm.at[src_v], rows_v, sem).wait()
        pltpu.sync_copy(rows_v, agg_sh.at[dst_v], add=True)

    _run_edge_loop(c, s, chunk_body)
    plsc.subcore_barrier()

    def writeback(k, carry):
        off = s * ROWS_PT + k * CHUNK
        pltpu.sync_copy(agg_sh.at[pl.ds(off, CHUNK)], rows_v)
        pltpu.sync_copy(rows_v, out_hbm.at[pl.ds(c * NPAD + off, CHUNK)])
        return carry

    lax.fori_loop(0, ROWS_PT // CHUNK, writeback, 0)
    pltpu.sync_copy(agg_sh.at[pl.ds(toff, tail)], rows_v.at[pl.ds(0, tail)])
    pltpu.sync_copy(rows_v.at[pl.ds(0, tail)], out_hbm.at[pl.ds(c * NPAD + toff, tail)])


_agg_call = pl.kernel(
    _agg_body,
    out_type=jax.ShapeDtypeStruct((NC * NPAD, D), jnp.float32),
    mesh=_MESH,
    scratch_types=[
        pltpu.VMEM((CHUNK,), jnp.int32),
        pltpu.VMEM((CHUNK,), jnp.int32),
        pltpu.VMEM((CHUNK, D), jnp.float32),
        pltpu.VMEM_SHARED((NPAD, D), jnp.float32),
        pltpu.SemaphoreType.DMA,
    ],
)


def _lin_body(x_ref, w_ref, d0_ref, d1_ref, hn_ref):
    deg = d0_ref[...] + d1_ref[...] + 1.0
    norm = lax.rsqrt(deg)
    h = lax.dot_general(
        x_ref[...], w_ref[...], (((1,), (1,)), ((), ())),
        preferred_element_type=jnp.float32,
    )
    hn_ref[...] = h * norm


ROWS_B = 1000

_lin_call = pl.pallas_call(
    _lin_body,
    grid=(N // ROWS_B,),
    in_specs=[
        pl.BlockSpec((ROWS_B, D), lambda i: (i, 0)),
        pl.BlockSpec((D, D), lambda i: (0, 0)),
        pl.BlockSpec((ROWS_B, 1), lambda i: (i, 0)),
        pl.BlockSpec((ROWS_B, 1), lambda i: (i, 0)),
    ],
    out_specs=pl.BlockSpec((ROWS_B, D), lambda i: (i, 0)),
    out_shape=jax.ShapeDtypeStruct((N, D), jnp.float32),
)


def _ln_body(p0_ref, p1_ref, hn_ref, x_ref, d0_ref, d1_ref, g_ref, b_ref, o_ref):
    deg = d0_ref[...] + d1_ref[...] + 1.0
    norm = lax.rsqrt(deg)
    agg = (p0_ref[...] + p1_ref[...] - hn_ref[...]) * norm
    h = agg + x_ref[...]
    mean = jnp.mean(h, axis=-1, keepdims=True)
    cent = h - mean
    var = jnp.mean(cent * cent, axis=-1, keepdims=True)
    hln = cent * lax.rsqrt(var + EPS) * g_ref[0:1, :] + b_ref[0:1, :]
    o_ref[...] = jnp.maximum(hln, 0.0)


_ln_call = pl.pallas_call(
    _ln_body,
    grid=(N // ROWS_B,),
    in_specs=[
        pl.BlockSpec((ROWS_B, D), lambda i: (i, 0)),
        pl.BlockSpec((ROWS_B, D), lambda i: (i, 0)),
        pl.BlockSpec((ROWS_B, D), lambda i: (i, 0)),
        pl.BlockSpec((ROWS_B, D), lambda i: (i, 0)),
        pl.BlockSpec((ROWS_B, 1), lambda i: (i, 0)),
        pl.BlockSpec((ROWS_B, 1), lambda i: (i, 0)),
        pl.BlockSpec((8, D), lambda i: (0, 0)),
        pl.BlockSpec((8, D), lambda i: (0, 0)),
    ],
    out_specs=pl.BlockSpec((ROWS_B, D), lambda i: (i, 0)),
    out_shape=jax.ShapeDtypeStruct((N, D), jnp.float32),
)


@jax.jit
def kernel(x, edge_index, W, ln_gamma, ln_beta):
    ei = edge_index.astype(jnp.int32)
    pad = jnp.full((E_PAD - E,), N, jnp.int32)
    src_p = jnp.concatenate([ei[0], pad])
    dst_p = jnp.concatenate([ei[1], pad])

    ones_c = jnp.concatenate([
        jnp.ones((CHUNK, D), jnp.float32),
        jnp.zeros((CHUNK, D), jnp.float32),
    ])
    deg_parts = _deg_call(dst_p, ones_c)
    d0 = deg_parts[:N, 0:1]
    d1 = deg_parts[NPAD:NPAD + N, 0:1]

    hn = _lin_call(x, W, d0, d1)
    hn_pad = jnp.concatenate([hn, jnp.zeros((NPAD - N, D), jnp.float32)])

    parts = _agg_call(hn_pad, src_p, dst_p)

    g8 = jnp.broadcast_to(ln_gamma.reshape(1, D), (8, D))
    b8 = jnp.broadcast_to(ln_beta.reshape(1, D), (8, D))
    return _ln_call(parts[:N], parts[NPAD:NPAD + N], hn, x, d0, d1, g8, b8)


# speedup vs baseline: 1.8484x; 1.6469x over previous
import functools

import jax
import jax.numpy as jnp
from jax import lax
from jax.experimental import pallas as pl
from jax.experimental.pallas import tpu as pltpu
from jax.experimental.pallas import tpu_sc as plsc

N = 10000
D = 128
E = 320000
EPS = 1e-5
NC, NS = 2, 16
NTILES = NC * NS
CHUNK = 128
CPT = 80
EPT = CPT * CHUNK
E_PAD = NTILES * EPT
NPAD = 10112
ROWS_PT = NPAD // NS

_MESH = plsc.VectorSubcoreMesh(core_axis_name="c", subcore_axis_name="s")


def _run_edge_loop(c, s, body_for_chunk):
    base = (c * NS + s) * EPT

    def step(j, carry):
        body_for_chunk(base + j * CHUNK)
        return carry

    lax.fori_loop(0, CPT, step, 0)


def _deg_body(dst_hbm, ones_hbm, out_hbm, dst_v, ones_v, zbuf_v, deg_sh):
    c = lax.axis_index("c")
    s = lax.axis_index("s")

    pltpu.sync_copy(ones_hbm.at[pl.ds(0, CHUNK)], ones_v)

    def zloop(k, carry):
        off = s * ROWS_PT + k * CHUNK
        pltpu.sync_copy(zbuf_v, deg_sh.at[pl.ds(off, CHUNK)])
        return carry

    pltpu.sync_copy(ones_hbm.at[pl.ds(CHUNK, CHUNK)], zbuf_v)
    lax.fori_loop(0, ROWS_PT // CHUNK, zloop, 0)
    tail = ROWS_PT % CHUNK
    toff = s * ROWS_PT + (ROWS_PT // CHUNK) * CHUNK
    pltpu.sync_copy(zbuf_v.at[pl.ds(0, tail)], deg_sh.at[pl.ds(toff, tail)])
    plsc.subcore_barrier()

    def chunk_body(eoff):
        pltpu.sync_copy(dst_hbm.at[pl.ds(eoff, CHUNK)], dst_v)
        pltpu.sync_copy(ones_v, deg_sh.at[dst_v], add=True)

    _run_edge_loop(c, s, chunk_body)
    plsc.subcore_barrier()

    def writeback(k, carry):
        off = s * ROWS_PT + k * CHUNK
        pltpu.sync_copy(deg_sh.at[pl.ds(off, CHUNK)], zbuf_v)
        pltpu.sync_copy(zbuf_v, out_hbm.at[pl.ds(c * NPAD + off, CHUNK)])
        return carry

    lax.fori_loop(0, ROWS_PT // CHUNK, writeback, 0)
    pltpu.sync_copy(deg_sh.at[pl.ds(toff, tail)], zbuf_v.at[pl.ds(0, tail)])
    pltpu.sync_copy(zbuf_v.at[pl.ds(0, tail)], out_hbm.at[pl.ds(c * NPAD + toff, tail)])


_deg_call = pl.kernel(
    _deg_body,
    out_type=jax.ShapeDtypeStruct((NC * NPAD, D), jnp.float32),
    mesh=_MESH,
    scratch_types=[
        pltpu.VMEM((CHUNK,), jnp.int32),
        pltpu.VMEM((CHUNK, D), jnp.float32),
        pltpu.VMEM((CHUNK, D), jnp.float32),
        pltpu.VMEM_SHARED((NPAD, D), jnp.float32),
    ],
)


def _agg_body(hn_hbm, src_hbm, dst_hbm, out_hbm, src_v, dst_v, rows_v,
              agg_sh, sem):
    c = lax.axis_index("c")
    s = lax.axis_index("s")

    def stage(k, carry):
        off = s * ROWS_PT + k * CHUNK
        pltpu.sync_copy(hn_hbm.at[pl.ds(off, CHUNK)], rows_v)
        pltpu.sync_copy(rows_v, agg_sh.at[pl.ds(off, CHUNK)])
        return carry

    lax.fori_loop(0, ROWS_PT // CHUNK, stage, 0)
    tail = ROWS_PT % CHUNK
    toff = s * ROWS_PT + (ROWS_PT // CHUNK) * CHUNK
    pltpu.sync_copy(hn_hbm.at[pl.ds(toff, tail)], rows_v.at[pl.ds(0, tail)])
    pltpu.sync_copy(rows_v.at[pl.ds(0, tail)], agg_sh.at[pl.ds(toff, tail)])
    plsc.subcore_barrier()

    def chunk_body(eoff):
        pltpu.sync_copy(src_hbm.at[pl.ds(eoff, CHUNK)], src_v)
        pltpu.sync_copy(dst_hbm.at[pl.ds(eoff, CHUNK)], dst_v)
        pltpu.async_copy(hn_hbm.at[src_v], rows_v, sem).wait()
        pltpu.sync_copy(rows_v, agg_sh.at[dst_v], add=True)

    _run_edge_loop(c, s, chunk_body)
    plsc.subcore_barrier()

    def writeback(k, carry):
        off = s * ROWS_PT + k * CHUNK
        pltpu.sync_copy(agg_sh.at[pl.ds(off, CHUNK)], rows_v)
        pltpu.sync_copy(rows_v, out_hbm.at[pl.ds(c * NPAD + off, CHUNK)])
        return carry

    lax.fori_loop(0, ROWS_PT // CHUNK, writeback, 0)
    pltpu.sync_copy(agg_sh.at[pl.ds(toff, tail)], rows_v.at[pl.ds(0, tail)])
    pltpu.sync_copy(rows_v.at[pl.ds(0, tail)], out_hbm.at[pl.ds(c * NPAD + toff, tail)])


_agg_call = pl.kernel(
    _agg_body,
    out_type=jax.ShapeDtypeStruct((NC * NPAD, D), jnp.float32),
    mesh=_MESH,
    scratch_types=[
        pltpu.VMEM((CHUNK,), jnp.int32),
        pltpu.VMEM((CHUNK,), jnp.int32),
        pltpu.VMEM((CHUNK, D), jnp.float32),
        pltpu.VMEM_SHARED((NPAD, D), jnp.float32),
        pltpu.SemaphoreType.DMA,
    ],
)


def _lin_body(x_ref, w_ref, d0_ref, d1_ref, hn_ref):
    deg = d0_ref[...] + d1_ref[...] + 1.0
    norm = lax.rsqrt(deg)
    h = lax.dot_general(
        x_ref[...], w_ref[...], (((1,), (1,)), ((), ())),
        preferred_element_type=jnp.float32,
    )
    hn_ref[...] = h * norm


ROWS_B = 1000

_lin_call = pl.pallas_call(
    _lin_body,
    grid=(N // ROWS_B,),
    in_specs=[
        pl.BlockSpec((ROWS_B, D), lambda i: (i, 0)),
        pl.BlockSpec((D, D), lambda i: (0, 0)),
        pl.BlockSpec((ROWS_B, 1), lambda i: (i, 0)),
        pl.BlockSpec((ROWS_B, 1), lambda i: (i, 0)),
    ],
    out_specs=pl.BlockSpec((ROWS_B, D), lambda i: (i, 0)),
    out_shape=jax.ShapeDtypeStruct((N, D), jnp.float32),
)


def _ln_body(p0_ref, p1_ref, hn_ref, x_ref, d0_ref, d1_ref, g_ref, b_ref, o_ref):
    deg = d0_ref[...] + d1_ref[...] + 1.0
    norm = lax.rsqrt(deg)
    agg = (p0_ref[...] + p1_ref[...] - hn_ref[...]) * norm
    h = agg + x_ref[...]
    mean = jnp.mean(h, axis=-1, keepdims=True)
    cent = h - mean
    var = jnp.mean(cent * cent, axis=-1, keepdims=True)
    hln = cent * lax.rsqrt(var + EPS) * g_ref[0:1, :] + b_ref[0:1, :]
    o_ref[...] = jnp.maximum(hln, 0.0)


_ln_call = pl.pallas_call(
    _ln_body,
    grid=(N // ROWS_B,),
    in_specs=[
        pl.BlockSpec((ROWS_B, D), lambda i: (i, 0)),
        pl.BlockSpec((ROWS_B, D), lambda i: (i, 0)),
        pl.BlockSpec((ROWS_B, D), lambda i: (i, 0)),
        pl.BlockSpec((ROWS_B, D), lambda i: (i, 0)),
        pl.BlockSpec((ROWS_B, 1), lambda i: (i, 0)),
        pl.BlockSpec((ROWS_B, 1), lambda i: (i, 0)),
        pl.BlockSpec((8, D), lambda i: (0, 0)),
        pl.BlockSpec((8, D), lambda i: (0, 0)),
    ],
    out_specs=pl.BlockSpec((ROWS_B, D), lambda i: (i, 0)),
    out_shape=jax.ShapeDtypeStruct((N, D), jnp.float32),
)


@jax.jit
def kernel(x, edge_index, W, ln_gamma, ln_beta):
    ei = edge_index.astype(jnp.int32)
    pad_i = jnp.arange(E_PAD - E, dtype=jnp.int32)
    src_p = jnp.concatenate([ei[0], pad_i % N])
    dst_p = jnp.concatenate([ei[1], N + pad_i % (NPAD - N)])

    ones_c = jnp.concatenate([
        jnp.ones((CHUNK, D), jnp.float32),
        jnp.zeros((CHUNK, D), jnp.float32),
    ])
    deg_parts = _deg_call(dst_p, ones_c)
    d0 = deg_parts[:N, 0:1]
    d1 = deg_parts[NPAD:NPAD + N, 0:1]

    hn = _lin_call(x, W, d0, d1)
    hn_pad = jnp.concatenate([hn, jnp.zeros((NPAD - N, D), jnp.float32)])

    parts = _agg_call(hn_pad, src_p, dst_p)

    g8 = jnp.broadcast_to(ln_gamma.reshape(1, D), (8, D))
    b8 = jnp.broadcast_to(ln_beta.reshape(1, D), (8, D))
    return _ln_call(parts[:N], parts[NPAD:NPAD + N], hn, x, d0, d1, g8, b8)


# speedup vs baseline: 2.3851x; 1.2903x over previous
import functools

import jax
import jax.numpy as jnp
from jax import lax
from jax.experimental import pallas as pl
from jax.experimental.pallas import tpu as pltpu
from jax.experimental.pallas import tpu_sc as plsc

N = 10000
D = 128
E = 320000
EPS = 1e-5
NC, NS = 2, 16
NTILES = NC * NS
CHUNK = 128
CPT = 80
EPT = CPT * CHUNK
E_PAD = NTILES * EPT
NPAD = 10112
ROWS_PT = NPAD // NS

_MESH = plsc.VectorSubcoreMesh(core_axis_name="c", subcore_axis_name="s")


def _run_edge_loop(c, s, body_for_chunk):
    base = (c * NS + s) * EPT

    def step(j, carry):
        body_for_chunk(base + j * CHUNK)
        return carry

    lax.fori_loop(0, CPT, step, 0)


def _deg_body(dst_hbm, ones_hbm, out_hbm, dst_v, ones_v, zbuf_v, deg_sh):
    c = lax.axis_index("c")
    s = lax.axis_index("s")

    pltpu.sync_copy(ones_hbm.at[pl.ds(0, CHUNK)], ones_v)

    def zloop(k, carry):
        off = s * ROWS_PT + k * CHUNK
        pltpu.sync_copy(zbuf_v, deg_sh.at[pl.ds(off, CHUNK)])
        return carry

    pltpu.sync_copy(ones_hbm.at[pl.ds(CHUNK, CHUNK)], zbuf_v)
    lax.fori_loop(0, ROWS_PT // CHUNK, zloop, 0)
    tail = ROWS_PT % CHUNK
    toff = s * ROWS_PT + (ROWS_PT // CHUNK) * CHUNK
    pltpu.sync_copy(zbuf_v.at[pl.ds(0, tail)], deg_sh.at[pl.ds(toff, tail)])
    plsc.subcore_barrier()

    def chunk_body(eoff):
        pltpu.sync_copy(dst_hbm.at[pl.ds(eoff, CHUNK)], dst_v)
        pltpu.sync_copy(ones_v, deg_sh.at[dst_v], add=True)

    _run_edge_loop(c, s, chunk_body)
    plsc.subcore_barrier()

    def writeback(k, carry):
        off = s * ROWS_PT + k * CHUNK
        pltpu.sync_copy(deg_sh.at[pl.ds(off, CHUNK)], zbuf_v)
        pltpu.sync_copy(zbuf_v, out_hbm.at[pl.ds(c * NPAD + off, CHUNK)])
        return carry

    lax.fori_loop(0, ROWS_PT // CHUNK, writeback, 0)
    pltpu.sync_copy(deg_sh.at[pl.ds(toff, tail)], zbuf_v.at[pl.ds(0, tail)])
    pltpu.sync_copy(zbuf_v.at[pl.ds(0, tail)], out_hbm.at[pl.ds(c * NPAD + toff, tail)])


_deg_call = pl.kernel(
    _deg_body,
    out_type=jax.ShapeDtypeStruct((NC * NPAD, D), jnp.float32),
    mesh=_MESH,
    scratch_types=[
        pltpu.VMEM((CHUNK,), jnp.int32),
        pltpu.VMEM((CHUNK, D), jnp.float32),
        pltpu.VMEM((CHUNK, D), jnp.float32),
        pltpu.VMEM_SHARED((NPAD, D), jnp.float32),
    ],
)


def _agg_body(hn_hbm, src_hbm, dst_hbm, out_hbm,
              src_v0, src_v1, dst_v0, dst_v1, rows_v0, rows_v1,
              agg_sh, sem0, sem1):
    c = lax.axis_index("c")
    s = lax.axis_index("s")
    rows_v = rows_v0

    def stage(k, carry):
        off = s * ROWS_PT + k * CHUNK
        pltpu.sync_copy(hn_hbm.at[pl.ds(off, CHUNK)], rows_v)
        pltpu.sync_copy(rows_v, agg_sh.at[pl.ds(off, CHUNK)])
        return carry

    lax.fori_loop(0, ROWS_PT // CHUNK, stage, 0)
    tail = ROWS_PT % CHUNK
    toff = s * ROWS_PT + (ROWS_PT // CHUNK) * CHUNK
    pltpu.sync_copy(hn_hbm.at[pl.ds(toff, tail)], rows_v.at[pl.ds(0, tail)])
    pltpu.sync_copy(rows_v.at[pl.ds(0, tail)], agg_sh.at[pl.ds(toff, tail)])
    plsc.subcore_barrier()

    base = (c * NS + s) * EPT
    srcs = (src_v0, src_v1)
    dsts = (dst_v0, dst_v1)
    rows = (rows_v0, rows_v1)
    sems = (sem0, sem1)

    pltpu.sync_copy(src_hbm.at[pl.ds(base, CHUNK)], src_v0)
    pltpu.sync_copy(dst_hbm.at[pl.ds(base, CHUNK)], dst_v0)
    pltpu.async_copy(hn_hbm.at[src_v0], rows_v0, sem0)

    def ring_step(g, carry):
        for b in (0, 1):
            j = 2 * g + b
            nxt = j + 1

            @pl.when(nxt < CPT)
            def _():
                off = base + nxt * CHUNK
                pltpu.sync_copy(src_hbm.at[pl.ds(off, CHUNK)], srcs[1 - b])
                pltpu.sync_copy(dst_hbm.at[pl.ds(off, CHUNK)], dsts[1 - b])
                pltpu.async_copy(hn_hbm.at[srcs[1 - b]], rows[1 - b], sems[1 - b])

            pltpu.make_async_copy(hn_hbm.at[srcs[b]], rows[b], sems[b]).wait()
            pltpu.sync_copy(rows[b], agg_sh.at[dsts[b]], add=True)
        return carry

    lax.fori_loop(0, CPT // 2, ring_step, 0)
    plsc.subcore_barrier()

    def writeback(k, carry):
        off = s * ROWS_PT + k * CHUNK
        pltpu.sync_copy(agg_sh.at[pl.ds(off, CHUNK)], rows_v)
        pltpu.sync_copy(rows_v, out_hbm.at[pl.ds(c * NPAD + off, CHUNK)])
        return carry

    lax.fori_loop(0, ROWS_PT // CHUNK, writeback, 0)
    pltpu.sync_copy(agg_sh.at[pl.ds(toff, tail)], rows_v.at[pl.ds(0, tail)])
    pltpu.sync_copy(rows_v.at[pl.ds(0, tail)], out_hbm.at[pl.ds(c * NPAD + toff, tail)])


_agg_call = pl.kernel(
    _agg_body,
    out_type=jax.ShapeDtypeStruct((NC * NPAD, D), jnp.float32),
    mesh=_MESH,
    scratch_types=[
        pltpu.VMEM((CHUNK,), jnp.int32),
        pltpu.VMEM((CHUNK,), jnp.int32),
        pltpu.VMEM((CHUNK,), jnp.int32),
        pltpu.VMEM((CHUNK,), jnp.int32),
        pltpu.VMEM((CHUNK, D), jnp.float32),
        pltpu.VMEM((CHUNK, D), jnp.float32),
        pltpu.VMEM_SHARED((NPAD, D), jnp.float32),
        pltpu.SemaphoreType.DMA,
        pltpu.SemaphoreType.DMA,
    ],
)


def _lin_body(x_ref, w_ref, d0_ref, d1_ref, hn_ref):
    deg = d0_ref[...] + d1_ref[...] + 1.0
    norm = lax.rsqrt(deg)
    h = lax.dot_general(
        x_ref[...], w_ref[...], (((1,), (1,)), ((), ())),
        preferred_element_type=jnp.float32,
    )
    hn_ref[...] = h * norm


ROWS_B = 1000

_lin_call = pl.pallas_call(
    _lin_body,
    grid=(N // ROWS_B,),
    in_specs=[
        pl.BlockSpec((ROWS_B, D), lambda i: (i, 0)),
        pl.BlockSpec((D, D), lambda i: (0, 0)),
        pl.BlockSpec((ROWS_B, 1), lambda i: (i, 0)),
        pl.BlockSpec((ROWS_B, 1), lambda i: (i, 0)),
    ],
    out_specs=pl.BlockSpec((ROWS_B, D), lambda i: (i, 0)),
    out_shape=jax.ShapeDtypeStruct((N, D), jnp.float32),
)


def _ln_body(p0_ref, p1_ref, hn_ref, x_ref, d0_ref, d1_ref, g_ref, b_ref, o_ref):
    deg = d0_ref[...] + d1_ref[...] + 1.0
    norm = lax.rsqrt(deg)
    agg = (p0_ref[...] + p1_ref[...] - hn_ref[...]) * norm
    h = agg + x_ref[...]
    mean = jnp.mean(h, axis=-1, keepdims=True)
    cent = h - mean
    var = jnp.mean(cent * cent, axis=-1, keepdims=True)
    hln = cent * lax.rsqrt(var + EPS) * g_ref[0:1, :] + b_ref[0:1, :]
    o_ref[...] = jnp.maximum(hln, 0.0)


_ln_call = pl.pallas_call(
    _ln_body,
    grid=(N // ROWS_B,),
    in_specs=[
        pl.BlockSpec((ROWS_B, D), lambda i: (i, 0)),
        pl.BlockSpec((ROWS_B, D), lambda i: (i, 0)),
        pl.BlockSpec((ROWS_B, D), lambda i: (i, 0)),
        pl.BlockSpec((ROWS_B, D), lambda i: (i, 0)),
        pl.BlockSpec((ROWS_B, 1), lambda i: (i, 0)),
        pl.BlockSpec((ROWS_B, 1), lambda i: (i, 0)),
        pl.BlockSpec((8, D), lambda i: (0, 0)),
        pl.BlockSpec((8, D), lambda i: (0, 0)),
    ],
    out_specs=pl.BlockSpec((ROWS_B, D), lambda i: (i, 0)),
    out_shape=jax.ShapeDtypeStruct((N, D), jnp.float32),
)


@jax.jit
def kernel(x, edge_index, W, ln_gamma, ln_beta):
    ei = edge_index.astype(jnp.int32)
    pad_i = jnp.arange(E_PAD - E, dtype=jnp.int32)
    src_p = jnp.concatenate([ei[0], pad_i % N])
    dst_p = jnp.concatenate([ei[1], N + pad_i % (NPAD - N)])

    ones_c = jnp.concatenate([
        jnp.ones((CHUNK, D), jnp.float32),
        jnp.zeros((CHUNK, D), jnp.float32),
    ])
    deg_parts = _deg_call(dst_p, ones_c)
    d0 = deg_parts[:N, 0:1]
    d1 = deg_parts[NPAD:NPAD + N, 0:1]

    hn = _lin_call(x, W, d0, d1)
    hn_pad = jnp.concatenate([hn, jnp.zeros((NPAD - N, D), jnp.float32)])

    parts = _agg_call(hn_pad, src_p, dst_p)

    g8 = jnp.broadcast_to(ln_gamma.reshape(1, D), (8, D))
    b8 = jnp.broadcast_to(ln_beta.reshape(1, D), (8, D))
    return _ln_call(parts[:N], parts[NPAD:NPAD + N], hn, x, d0, d1, g8, b8)


# speedup vs baseline: 2.6871x; 1.1266x over previous
import functools

import jax
import jax.numpy as jnp
from jax import lax
from jax.experimental import pallas as pl
from jax.experimental.pallas import tpu as pltpu
from jax.experimental.pallas import tpu_sc as plsc

N = 10000
D = 128
E = 320000
EPS = 1e-5
NC, NS = 2, 16
NTILES = NC * NS
CHUNK = 128
CPT = 80
EPT = CPT * CHUNK
E_PAD = NTILES * EPT
NPAD = 10112
ROWS_PT = NPAD // NS

_MESH = plsc.VectorSubcoreMesh(core_axis_name="c", subcore_axis_name="s")


def _run_edge_loop(c, s, body_for_chunk):
    base = (c * NS + s) * EPT

    def step(j, carry):
        body_for_chunk(base + j * CHUNK)
        return carry

    lax.fori_loop(0, CPT, step, 0)


def _deg_body(dst_hbm, ones_hbm, out_hbm, dst_v0, dst_v1, ones_v, zbuf_v,
              deg_sh, dsem0, dsem1):
    c = lax.axis_index("c")
    s = lax.axis_index("s")

    pltpu.sync_copy(ones_hbm.at[pl.ds(0, CHUNK)], ones_v)

    def zloop(k, carry):
        off = s * ROWS_PT + k * CHUNK
        pltpu.sync_copy(zbuf_v, deg_sh.at[pl.ds(off, CHUNK)])
        return carry

    pltpu.sync_copy(ones_hbm.at[pl.ds(CHUNK, CHUNK)], zbuf_v)
    lax.fori_loop(0, ROWS_PT // CHUNK, zloop, 0)
    tail = ROWS_PT % CHUNK
    toff = s * ROWS_PT + (ROWS_PT // CHUNK) * CHUNK
    pltpu.sync_copy(zbuf_v.at[pl.ds(0, tail)], deg_sh.at[pl.ds(toff, tail)])
    plsc.subcore_barrier()

    base = (c * NS + s) * EPT
    dsts = (dst_v0, dst_v1)
    dsems = (dsem0, dsem1)

    pltpu.async_copy(dst_hbm.at[pl.ds(base, CHUNK)], dst_v0, dsem0)

    def ring_step(g, carry):
        for b in (0, 1):
            j = 2 * g + b
            nxt = j + 1

            @pl.when(nxt < CPT)
            def _():
                off = base + nxt * CHUNK
                pltpu.async_copy(dst_hbm.at[pl.ds(off, CHUNK)], dsts[1 - b],
                                 dsems[1 - b])

            pltpu.make_async_copy(dst_hbm.at[pl.ds(base, CHUNK)], dsts[b],
                                  dsems[b]).wait()
            pltpu.sync_copy(ones_v, deg_sh.at[dsts[b]], add=True)
        return carry

    lax.fori_loop(0, CPT // 2, ring_step, 0)
    plsc.subcore_barrier()

    def writeback(k, carry):
        off = s * ROWS_PT + k * CHUNK
        pltpu.sync_copy(deg_sh.at[pl.ds(off, CHUNK)], zbuf_v)
        pltpu.sync_copy(zbuf_v, out_hbm.at[pl.ds(c * NPAD + off, CHUNK)])
        return carry

    lax.fori_loop(0, ROWS_PT // CHUNK, writeback, 0)
    pltpu.sync_copy(deg_sh.at[pl.ds(toff, tail)], zbuf_v.at[pl.ds(0, tail)])
    pltpu.sync_copy(zbuf_v.at[pl.ds(0, tail)], out_hbm.at[pl.ds(c * NPAD + toff, tail)])


_deg_call = pl.kernel(
    _deg_body,
    out_type=jax.ShapeDtypeStruct((NC * NPAD, D), jnp.float32),
    mesh=_MESH,
    scratch_types=[
        pltpu.VMEM((CHUNK,), jnp.int32),
        pltpu.VMEM((CHUNK,), jnp.int32),
        pltpu.VMEM((CHUNK, D), jnp.float32),
        pltpu.VMEM((CHUNK, D), jnp.float32),
        pltpu.VMEM_SHARED((NPAD, D), jnp.float32),
        pltpu.SemaphoreType.DMA,
        pltpu.SemaphoreType.DMA,
    ],
)


def _agg_body(hn_hbm, src_hbm, dst_hbm, out_hbm,
              src_v0, src_v1, dst_v0, dst_v1, rows_v0, rows_v1,
              agg_sh, sem0, sem1):
    c = lax.axis_index("c")
    s = lax.axis_index("s")
    rows_v = rows_v0

    def stage(k, carry):
        off = s * ROWS_PT + k * CHUNK
        pltpu.sync_copy(hn_hbm.at[pl.ds(off, CHUNK)], rows_v)
        pltpu.sync_copy(rows_v, agg_sh.at[pl.ds(off, CHUNK)])
        return carry

    lax.fori_loop(0, ROWS_PT // CHUNK, stage, 0)
    tail = ROWS_PT % CHUNK
    toff = s * ROWS_PT + (ROWS_PT // CHUNK) * CHUNK
    pltpu.sync_copy(hn_hbm.at[pl.ds(toff, tail)], rows_v.at[pl.ds(0, tail)])
    pltpu.sync_copy(rows_v.at[pl.ds(0, tail)], agg_sh.at[pl.ds(toff, tail)])
    plsc.subcore_barrier()

    base = (c * NS + s) * EPT
    srcs = (src_v0, src_v1)
    dsts = (dst_v0, dst_v1)
    rows = (rows_v0, rows_v1)
    sems = (sem0, sem1)

    pltpu.sync_copy(src_hbm.at[pl.ds(base, CHUNK)], src_v0)
    pltpu.sync_copy(dst_hbm.at[pl.ds(base, CHUNK)], dst_v0)
    pltpu.async_copy(hn_hbm.at[src_v0], rows_v0, sem0)

    def ring_step(g, carry):
        for b in (0, 1):
            j = 2 * g + b
            nxt = j + 1

            @pl.when(nxt < CPT)
            def _():
                off = base + nxt * CHUNK
                pltpu.sync_copy(src_hbm.at[pl.ds(off, CHUNK)], srcs[1 - b])
                pltpu.sync_copy(dst_hbm.at[pl.ds(off, CHUNK)], dsts[1 - b])
                pltpu.async_copy(hn_hbm.at[srcs[1 - b]], rows[1 - b], sems[1 - b])

            pltpu.make_async_copy(hn_hbm.at[srcs[b]], rows[b], sems[b]).wait()
            pltpu.sync_copy(rows[b], agg_sh.at[dsts[b]], add=True)
        return carry

    lax.fori_loop(0, CPT // 2, ring_step, 0)
    plsc.subcore_barrier()

    def writeback(k, carry):
        off = s * ROWS_PT + k * CHUNK
        pltpu.sync_copy(agg_sh.at[pl.ds(off, CHUNK)], rows_v)
        pltpu.sync_copy(rows_v, out_hbm.at[pl.ds(c * NPAD + off, CHUNK)])
        return carry

    lax.fori_loop(0, ROWS_PT // CHUNK, writeback, 0)
    pltpu.sync_copy(agg_sh.at[pl.ds(toff, tail)], rows_v.at[pl.ds(0, tail)])
    pltpu.sync_copy(rows_v.at[pl.ds(0, tail)], out_hbm.at[pl.ds(c * NPAD + toff, tail)])


_agg_call = pl.kernel(
    _agg_body,
    out_type=jax.ShapeDtypeStruct((NC * NPAD, D), jnp.float32),
    mesh=_MESH,
    scratch_types=[
        pltpu.VMEM((CHUNK,), jnp.int32),
        pltpu.VMEM((CHUNK,), jnp.int32),
        pltpu.VMEM((CHUNK,), jnp.int32),
        pltpu.VMEM((CHUNK,), jnp.int32),
        pltpu.VMEM((CHUNK, D), jnp.float32),
        pltpu.VMEM((CHUNK, D), jnp.float32),
        pltpu.VMEM_SHARED((NPAD, D), jnp.float32),
        pltpu.SemaphoreType.DMA,
        pltpu.SemaphoreType.DMA,
    ],
)


def _lin_body(x_ref, w_ref, d0_ref, d1_ref, hn_ref):
    deg = d0_ref[...] + d1_ref[...] + 1.0
    norm = lax.rsqrt(deg)
    h = lax.dot_general(
        x_ref[...], w_ref[...], (((1,), (1,)), ((), ())),
        preferred_element_type=jnp.float32,
    )
    hn_ref[...] = h * norm


ROWS_B = 1000

_lin_call = pl.pallas_call(
    _lin_body,
    grid=(N // ROWS_B,),
    in_specs=[
        pl.BlockSpec((ROWS_B, D), lambda i: (i, 0)),
        pl.BlockSpec((D, D), lambda i: (0, 0)),
        pl.BlockSpec((ROWS_B, 1), lambda i: (i, 0)),
        pl.BlockSpec((ROWS_B, 1), lambda i: (i, 0)),
    ],
    out_specs=pl.BlockSpec((ROWS_B, D), lambda i: (i, 0)),
    out_shape=jax.ShapeDtypeStruct((NPAD, D), jnp.float32),
)


def _ln_body(p0_ref, p1_ref, hn_ref, x_ref, d0_ref, d1_ref, g_ref, b_ref, o_ref):
    deg = d0_ref[...] + d1_ref[...] + 1.0
    norm = lax.rsqrt(deg)
    agg = (p0_ref[...] + p1_ref[...] - hn_ref[...]) * norm
    h = agg + x_ref[...]
    mean = jnp.mean(h, axis=-1, keepdims=True)
    cent = h - mean
    var = jnp.mean(cent * cent, axis=-1, keepdims=True)
    hln = cent * lax.rsqrt(var + EPS) * g_ref[0:1, :] + b_ref[0:1, :]
    o_ref[...] = jnp.maximum(hln, 0.0)


_ln_call = pl.pallas_call(
    _ln_body,
    grid=(N // ROWS_B,),
    in_specs=[
        pl.BlockSpec((ROWS_B, D), lambda i: (i, 0)),
        pl.BlockSpec((ROWS_B, D), lambda i: (i, 0)),
        pl.BlockSpec((ROWS_B, D), lambda i: (i, 0)),
        pl.BlockSpec((ROWS_B, D), lambda i: (i, 0)),
        pl.BlockSpec((ROWS_B, 1), lambda i: (i, 0)),
        pl.BlockSpec((ROWS_B, 1), lambda i: (i, 0)),
        pl.BlockSpec((8, D), lambda i: (0, 0)),
        pl.BlockSpec((8, D), lambda i: (0, 0)),
    ],
    out_specs=pl.BlockSpec((ROWS_B, D), lambda i: (i, 0)),
    out_shape=jax.ShapeDtypeStruct((N, D), jnp.float32),
)


@jax.jit
def kernel(x, edge_index, W, ln_gamma, ln_beta):
    ei = edge_index.astype(jnp.int32)
    pad_i = jnp.arange(E_PAD - E, dtype=jnp.int32)
    src_p = jnp.concatenate([ei[0], pad_i % N])
    dst_p = jnp.concatenate([ei[1], N + pad_i % (NPAD - N)])

    ones_c = jnp.concatenate([
        jnp.ones((CHUNK, D), jnp.float32),
        jnp.zeros((CHUNK, D), jnp.float32),
    ])
    deg_parts = _deg_call(dst_p, ones_c)
    d0 = deg_parts[:N, 0:1]
    d1 = deg_parts[NPAD:NPAD + N, 0:1]

    hn_pad = _lin_call(x, W, d0, d1)
    parts = _agg_call(hn_pad, src_p, dst_p)

    g8 = jnp.broadcast_to(ln_gamma.reshape(1, D), (8, D))
    b8 = jnp.broadcast_to(ln_beta.reshape(1, D), (8, D))
    return _ln_call(parts[:N], parts[NPAD:NPAD + N], hn_pad, x, d0, d1, g8, b8)
